# baseline (speedup 1.0000x reference)
import functools

import jax
import jax.numpy as jnp
from jax.experimental import pallas as pl
from jax.experimental.pallas import tpu as pltpu

_RATE = 0.8
_ALPHA = 0.8
_BETA = 1.2


def _sum_body(x_ref, o_ref):
    o_ref[...] = jnp.sum(x_ref[...], axis=1, keepdims=True)


def _mask_body(fv_ref, o_ref, *, k):
    fv = fv_ref[...]
    gt = (fv[:, None, :] > fv[:, :, None]).astype(jnp.float32)
    cnt = jnp.sum(gt, axis=2)
    o_ref[...] = jnp.where(cnt < k, _ALPHA, _BETA).astype(jnp.float32)


def kernel(inputs):
    b, c, h, w = inputs.shape
    n = h * w
    rows = b * c
    x = inputs.reshape(rows, n)
    k = int(c * _RATE)

    rb = 64
    sums = pl.pallas_call(
        _sum_body,
        grid=(rows // rb,),
        in_specs=[pl.BlockSpec((rb, n), lambda i: (i, 0))],
        out_specs=pl.BlockSpec((rb, 1), lambda i: (i, 0)),
        out_shape=jax.ShapeDtypeStruct((rows, 1), jnp.float32),
        compiler_params=pltpu.CompilerParams(
            dimension_semantics=("parallel",),
        ),
    )(x)

    fv = sums.reshape(b, c) * (1.0 / n)
    return pl.pallas_call(
        functools.partial(_mask_body, k=k),
        out_shape=jax.ShapeDtypeStruct((b, c), jnp.float32),
    )(fv)


# speedup vs baseline: 3.9950x; 3.9950x over previous
import functools

import jax
import jax.numpy as jnp
from jax.experimental import pallas as pl
from jax.experimental.pallas import tpu as pltpu

_RATE = 0.8
_ALPHA = 0.8
_BETA = 1.2


def _sum_body(x_ref, o_ref):
    s = jnp.sum(x_ref[...], axis=(2, 3))
    o_ref[...] = s[None, None]


def _mask_body(fv_ref, o_ref, *, k):
    fv = fv_ref[...]
    gt = (fv[:, None, :] > fv[:, :, None]).astype(jnp.float32)
    cnt = jnp.sum(gt, axis=2)
    o_ref[...] = jnp.where(cnt < k, _ALPHA, _BETA).astype(jnp.float32)


def kernel(inputs):
    b, c, h, w = inputs.shape
    n = h * w
    k = int(c * _RATE)

    cb = 64
    cg = c // cb
    sums = pl.pallas_call(
        _sum_body,
        grid=(b, cg),
        in_specs=[pl.BlockSpec((1, cb, h, w), lambda i, j: (i, j, 0, 0))],
        out_specs=pl.BlockSpec((1, 1, 1, cb), lambda i, j: (i, j, 0, 0)),
        out_shape=jax.ShapeDtypeStruct((b, cg, 1, cb), jnp.float32),
        compiler_params=pltpu.CompilerParams(
            dimension_semantics=("parallel", "parallel"),
        ),
    )(inputs)

    fv = sums.reshape(b, c) * (1.0 / n)
    return pl.pallas_call(
        functools.partial(_mask_body, k=k),
        out_shape=jax.ShapeDtypeStruct((b, c), jnp.float32),
    )(fv)


# speedup vs baseline: 4.0460x; 1.0128x over previous
import jax
import jax.numpy as jnp
from jax.experimental import pallas as pl
from jax.experimental.pallas import tpu as pltpu

_RATE = 0.8
_ALPHA = 0.8
_BETA = 1.2


def _body(x_ref, o_ref, acc_ref, *, k, inv_n):
    j = pl.program_id(1)
    nj = pl.num_programs(1)

    @pl.when(j == 0)
    def _():
        acc_ref[...] = jnp.zeros_like(acc_ref)

    acc_ref[...] += jnp.sum(x_ref[...], axis=(2, 3))

    @pl.when(j == nj - 1)
    def _():
        fv = acc_ref[...] * inv_n
        gt = (fv[:, None, :] > fv[:, :, None]).astype(jnp.float32)
        cnt = jnp.sum(gt, axis=2)
        o_ref[...] = jnp.where(cnt < k, _ALPHA, _BETA)[None].astype(
            jnp.float32
        )


def kernel(inputs):
    b, c, h, w = inputs.shape
    n = h * w
    k = int(c * _RATE)

    hb = 56
    steps = h // hb

    import functools

    body = functools.partial(_body, k=k, inv_n=1.0 / n)
    out = pl.pallas_call(
        body,
        grid=(b, steps),
        in_specs=[pl.BlockSpec((1, c, hb, w), lambda i, j: (i, 0, j, 0))],
        out_specs=pl.BlockSpec((1, 1, c), lambda i, j: (i, 0, 0)),
        out_shape=jax.ShapeDtypeStruct((b, 1, c), jnp.float32),
        scratch_shapes=[pltpu.VMEM((1, c), jnp.float32)],
        compiler_params=pltpu.CompilerParams(
            dimension_semantics=("parallel", "arbitrary"),
        ),
    )(inputs)
    return out.reshape(b, c)
